# baseline (speedup 1.0000x reference)
import functools

import jax
import jax.numpy as jnp
from jax import lax
from jax.experimental import pallas as pl
from jax.experimental.pallas import tpu as pltpu
from jax.experimental.pallas import tpu_sc as plsc

V = 1000000
D = 32
B = 16384

_info = plsc.get_sparse_core_info()
_NC, _NS = _info.num_cores, _info.num_subcores
_NW = _NC * _NS
_BPW = B // _NW
_CHUNK = 128
_NCHUNK = _BPW // _CHUNK
_HALF = _BPW // 2

_mesh = plsc.VectorSubcoreMesh(core_axis_name="c", subcore_axis_name="s")


@functools.partial(
    pl.kernel,
    mesh=_mesh,
    out_type=[
        jax.ShapeDtypeStruct((B, D), jnp.float32),
        jax.ShapeDtypeStruct((B, D), jnp.float32),
        jax.ShapeDtypeStruct((B, D), jnp.float32),
    ],
    scratch_types=[
        pltpu.VMEM((_NCHUNK, _CHUNK), jnp.int32),
        pltpu.VMEM((_NCHUNK, _CHUNK), jnp.int32),
        pltpu.VMEM((_HALF, 128), jnp.float32),
        pltpu.VMEM((_BPW, D), jnp.float32),
        pltpu.SemaphoreType.DMA,
    ],
    compiler_params=pltpu.CompilerParams(needs_layout_passes=False),
)
def _gather3(tq_hbm, sq_hbm, wh_hbm, wd_hbm, ww_hbm, oh_hbm, od_hbm, ow_hbm,
             tq_v, sq_v, rows_pad, rows_out, sem):
    wid = lax.axis_index("s") * _NC + lax.axis_index("c")
    base = wid * _BPW
    pltpu.sync_copy(tq_hbm.at[wid], tq_v)
    pltpu.sync_copy(sq_hbm.at[wid], sq_v)
    tabs = (wh_hbm, wd_hbm, ww_hbm)
    outs = (oh_hbm, od_hbm, ow_hbm)
    lane = lax.iota(jnp.int32, 16)

    for t in range(3):
        tab = tabs[t]
        for h in range(2):
            copies = [
                pltpu.async_copy(tab.at[tq_v.at[2 * h + jj]],
                                 rows_pad.at[pl.ds(jj * _CHUNK, _CHUNK)], sem)
                for jj in range(2)
            ]
            for c in copies:
                c.wait()

            def select(p, _):
                i = h * _HALF + p
                j16 = jnp.full((16,), i >> 7, jnp.int32)
                p16 = jnp.full((16,), i & 127, jnp.int32)
                pp16 = jnp.full((16,), p, jnp.int32)
                i16 = jnp.full((16,), i, jnp.int32)
                scol = plsc.load_gather(sq_v, [j16, p16]) * 32
                for half in range(2):
                    c16 = lane + 16 * half
                    val = plsc.load_gather(rows_pad, [pp16, scol + c16])
                    plsc.store_scatter(rows_out, [i16, c16], val)
                return 0

            lax.fori_loop(0, _HALF, select, 0)
        pltpu.sync_copy(rows_out, outs[t].at[pl.ds(base, _BPW)])


def kernel(index, W_hour, W_day, W_week):
    idx = index.astype(jnp.int32)
    tq = (idx >> 2).reshape(_NW, _NCHUNK, _CHUNK)
    sq = (idx & 3).reshape(_NW, _NCHUNK, _CHUNK)
    wph = W_hour.reshape(V // 4, 128)
    wpd = W_day.reshape(V // 4, 128)
    wpw = W_week.reshape(V // 4, 128)
    out = _gather3(tq, sq, wph, wpd, wpw)
    return tuple(out)


# speedup vs baseline: 1.4692x; 1.4692x over previous
import functools

import jax
import jax.numpy as jnp
from jax import lax
from jax.experimental import pallas as pl
from jax.experimental.pallas import tpu as pltpu
from jax.experimental.pallas import tpu_sc as plsc

V = 1000000
D = 32
B = 16384

_info = plsc.get_sparse_core_info()
_NC, _NS = _info.num_cores, _info.num_subcores
_NW = _NC * _NS
_BPW = B // _NW
_NB = 16
_NG = _BPW // _NB

_mesh = plsc.VectorSubcoreMesh(core_axis_name="c", subcore_axis_name="s")


@functools.partial(
    pl.kernel,
    mesh=_mesh,
    out_type=[
        jax.ShapeDtypeStruct((B, D), jnp.float32),
        jax.ShapeDtypeStruct((B, D), jnp.float32),
        jax.ShapeDtypeStruct((B, D), jnp.float32),
    ],
    scratch_types=[
        pltpu.VMEM((1, _BPW), jnp.int32),
        pltpu.SMEM((1, _BPW), jnp.int32),
        pltpu.VMEM((_BPW, D), jnp.float32),
        pltpu.SemaphoreType.DMA,
    ],
    compiler_params=pltpu.CompilerParams(needs_layout_passes=False),
)
def _gather3(idx_hbm, wh_hbm, wd_hbm, ww_hbm, oh_hbm, od_hbm, ow_hbm,
             idx_v, idx_s, rows, sem):
    wid = lax.axis_index("s") * _NC + lax.axis_index("c")
    base = wid * _BPW
    pltpu.sync_copy(idx_hbm.at[wid], idx_v)
    tabs = (wh_hbm, wd_hbm, ww_hbm)
    outs = (oh_hbm, od_hbm, ow_hbm)

    def extract(g, _):
        vec = idx_v[0, pl.ds(g * _NB, _NB)]
        for l in range(_NB):
            idx_s[0, g * _NB + l] = vec[l]
        return 0

    lax.fori_loop(0, _NG, extract, 0)

    for t in range(3):
        tab = tabs[t]

        def fire(g, _):
            gb = g * _NB
            for l in range(_NB):
                rid = idx_s[0, gb + l]
                pltpu.async_copy(tab.at[pl.ds(rid, 1)],
                                 rows.at[pl.ds(gb + l, 1)], sem)
            return 0

        def drain(g, _):
            for l in range(_NB):
                pltpu.make_async_copy(tab.at[pl.ds(0, 1)],
                                      rows.at[pl.ds(0, 1)], sem).wait()
            return 0

        lax.fori_loop(0, _NG, fire, 0)
        lax.fori_loop(0, _NG, drain, 0)
        pltpu.sync_copy(rows, outs[t].at[pl.ds(base, _BPW)])


def kernel(index, W_hour, W_day, W_week):
    idx = index.astype(jnp.int32).reshape(_NW, 1, _BPW)
    out = _gather3(idx, W_hour, W_day, W_week)
    return tuple(out)
